# baseline (speedup 1.0000x reference)
import functools

import jax
import jax.numpy as jnp
from jax import lax
from jax.experimental import pallas as pl
from jax.experimental.pallas import tpu as pltpu
from jax.experimental.pallas import tpu_sc as plsc

NC = 2
NS = 16
NW = NC * NS
L = 16

D = 64
G = 128
NBUF = 2


def _sc_gather_sum(table, idx, n_idx):
    per_w = n_idx // NW
    nchunk = per_w // G
    mesh = plsc.VectorSubcoreMesh(core_axis_name="c", subcore_axis_name="s")

    @functools.partial(
        pl.kernel,
        out_type=jax.ShapeDtypeStruct((NW, D), jnp.float32),
        mesh=mesh,
        scratch_types=[
            pltpu.VMEM((per_w,), jnp.int32),
            pltpu.VMEM((NBUF, G, D), jnp.float32),
            pltpu.VMEM((D,), jnp.float32),
            pltpu.SemaphoreType.DMA((NBUF,)),
        ],
        compiler_params=pltpu.CompilerParams(use_tc_tiling_on_sc=False),
    )
    def k(table_hbm, idx_hbm, out_hbm, idx_v, buf_v, acc_v, sems):
        wid = lax.axis_index("s") * NC + lax.axis_index("c")
        base = wid * per_w
        pltpu.sync_copy(idx_hbm.at[pl.ds(base, per_w)], idx_v)

        def start(c, slot):
            pltpu.make_async_copy(
                table_hbm.at[idx_v.at[pl.ds(c * G, G)]],
                buf_v.at[slot],
                sems.at[slot],
            ).start()

        def wait(slot):
            pltpu.make_async_copy(
                table_hbm.at[idx_v.at[pl.ds(0, G)]],
                buf_v.at[slot],
                sems.at[slot],
            ).wait()

        def row_body(i, a, slot):
            a0, a1, a2, a3 = a
            return (
                a0 + buf_v[slot, i, pl.ds(0, L)],
                a1 + buf_v[slot, i, pl.ds(L, L)],
                a2 + buf_v[slot, i, pl.ds(2 * L, L)],
                a3 + buf_v[slot, i, pl.ds(3 * L, L)],
            )

        for b in range(NBUF):
            start(b, b)

        def outer_body(co, carry):
            for b in range(NBUF):
                c = co * NBUF + b
                wait(b)
                carry = lax.fori_loop(
                    0, G, functools.partial(row_body, slot=b), carry, unroll=2
                )

                @pl.when(c + NBUF < nchunk)
                def _():
                    start(c + NBUF, b)

            return carry

        z = jnp.zeros((L,), jnp.float32)
        a0, a1, a2, a3 = lax.fori_loop(
            0, nchunk // NBUF, outer_body, (z, z, z, z)
        )
        acc_v[pl.ds(0, L)] = a0
        acc_v[pl.ds(L, L)] = a1
        acc_v[pl.ds(2 * L, L)] = a2
        acc_v[pl.ds(3 * L, L)] = a3
        pltpu.sync_copy(acc_v, out_hbm.at[wid])

    return k(table, idx)


def _tc_finish(partials, w, b, n_rows, n_idx):
    def body(p_ref, w_ref, b_ref, o_ref):
        s = jnp.sum(p_ref[...], axis=0, keepdims=True) * (1.0 / n_idx)
        y = lax.dot_general(
            s, w_ref[...], (((1,), (1,)), ((), ())),
            preferred_element_type=jnp.float32,
        ) + b_ref[...][None, :]
        y = jnp.where(y >= 0, y, 0.01 * y)
        o_ref[...] = jnp.broadcast_to(y, o_ref.shape)

    return pl.pallas_call(
        body,
        out_shape=jax.ShapeDtypeStruct((n_rows, w.shape[0]), jnp.float32),
    )(partials, w, b)


@jax.jit
def kernel(x, mp_neighbors, item_table, neigh_w, neigh_b, mp):
    flat_idx = mp_neighbors.reshape(-1)
    partials = _sc_gather_sum(item_table, flat_idx, flat_idx.shape[0])
    return _tc_finish(partials, neigh_w, neigh_b, x.shape[0], flat_idx.shape[0])
